# baseline (speedup 1.0000x reference)
import jax
import jax.numpy as jnp
from jax import lax
from jax.experimental import pallas as pl
from jax.experimental.pallas import tpu as pltpu
from jax.experimental.pallas import tpu_sc as plsc

N_NODES = 10000
N_EDGES = 320000
D_IN = 128
D_H = 64

NC = 2
NS = 16
NW = NC * NS
NP = 10240
CH = 128
BLK = 8
NBLK = 10
E_W_PAD = NBLK * BLK * CH
SLAB = NP // NS


def _seg_body(p_hbm, src_hbm, dst_hbm, zeros_hbm, s_out,
              shared_acc, src_v, dst_v, rows_a, rows_b,
              sem_ga, sem_gb, sem_sa, sem_sb, sem_i):
    cid = lax.axis_index("c")
    sid = lax.axis_index("s")
    w = cid * NS + sid
    slab = pl.multiple_of(sid * SLAB, 8)
    rows = (rows_a, rows_b)
    sem_g = (sem_ga, sem_gb)
    sem_s = (sem_sa, sem_sb)

    h1 = pltpu.async_copy(src_hbm.at[w, 0], src_v.at[0], sem_i)
    h2 = pltpu.async_copy(dst_hbm.at[w, 0], dst_v.at[0], sem_i)
    pltpu.sync_copy(zeros_hbm, shared_acc.at[pl.ds(slab, SLAB)])
    h1.wait()
    h2.wait()
    pltpu.async_copy(p_hbm.at[src_v.at[0, 0]], rows[0], sem_g[0])
    plsc.subcore_barrier()

    def _blk(blk, _):
        par = lax.rem(blk, 2)
        npar = 1 - par
        for b in range(BLK):
            pb = b % 2
            pltpu.make_async_copy(p_hbm.at[src_v.at[par, b]],
                                  rows[pb], sem_g[pb]).wait()
            pltpu.async_copy(rows[pb], shared_acc.at[dst_v.at[par, b]],
                             sem_s[pb], add=True)
            if b >= 1:
                pltpu.make_async_copy(zeros_hbm.at[pl.ds(0, CH)],
                                      rows[1 - pb], sem_s[1 - pb]).wait()
            else:
                @pl.when(blk > 0)
                def _w():
                    pltpu.make_async_copy(zeros_hbm.at[pl.ds(0, CH)],
                                          rows[1], sem_s[1]).wait()
            if b == 0:
                pltpu.async_copy(src_hbm.at[w, blk + 1], src_v.at[npar],
                                 sem_i)
                pltpu.async_copy(dst_hbm.at[w, blk + 1], dst_v.at[npar],
                                 sem_i)
            if b + 1 < BLK:
                pltpu.async_copy(p_hbm.at[src_v.at[par, b + 1]],
                                 rows[1 - pb], sem_g[1 - pb])
            else:
                pltpu.make_async_copy(src_hbm.at[w, blk + 1],
                                      src_v.at[npar], sem_i).wait()
                pltpu.make_async_copy(dst_hbm.at[w, blk + 1],
                                      dst_v.at[npar], sem_i).wait()
                pltpu.async_copy(p_hbm.at[src_v.at[npar, 0]],
                                 rows[0], sem_g[0])
        return 0

    lax.fori_loop(0, NBLK, _blk, 0)

    pltpu.make_async_copy(zeros_hbm.at[pl.ds(0, CH)], rows[1],
                          sem_s[1]).wait()
    pltpu.make_async_copy(p_hbm.at[src_v.at[0, 0]], rows[0],
                          sem_g[0]).wait()
    plsc.subcore_barrier()
    pltpu.sync_copy(shared_acc.at[pl.ds(slab, SLAB)],
                    s_out.at[cid, pl.ds(slab, SLAB)])


_seg_sum = pl.kernel(
    _seg_body,
    out_type=jax.ShapeDtypeStruct((NC, NP, 128), jnp.float32),
    mesh=plsc.VectorSubcoreMesh(core_axis_name="c", subcore_axis_name="s"),
    scratch_types=[
        pltpu.VMEM_SHARED((NP, 128), jnp.float32),
        pltpu.VMEM((2, BLK, CH), jnp.int32),
        pltpu.VMEM((2, BLK, CH), jnp.int32),
        pltpu.VMEM((CH, 128), jnp.float32),
        pltpu.VMEM((CH, 128), jnp.float32),
        pltpu.SemaphoreType.DMA,
        pltpu.SemaphoreType.DMA,
        pltpu.SemaphoreType.DMA,
        pltpu.SemaphoreType.DMA,
        pltpu.SemaphoreType.DMA,
    ],
    name="seg_sum",
)


ROW_BLK = 1000


def _proj1_body(x_ref, wl_ref, wr_ref, p_ref, r_ref):
    xb = x_ref[...]
    pw = jnp.dot(xb, wl_ref[...], preferred_element_type=jnp.float32)
    p_ref[...] = jnp.concatenate(
        [pw, jnp.ones((ROW_BLK, 128 - D_H), jnp.float32)], axis=1)
    r_ref[...] = jnp.dot(xb, wr_ref[...], preferred_element_type=jnp.float32)


def _combine1_proj2_body(s_ref, r_ref, b_ref, wl_ref, wr_ref,
                         p2_ref, r2_ref):
    cnt = s_ref[0, :, D_H:D_H + 1] + s_ref[1, :, D_H:D_H + 1]
    inv = 1.0 / jnp.maximum(cnt, 1.0)
    summed = s_ref[0, :, :D_H] + s_ref[1, :, :D_H]
    h = summed * inv + b_ref[...] + r_ref[...]
    h = jnp.maximum(h, 0.0)
    hw = jnp.dot(h, wl_ref[...], preferred_element_type=jnp.float32)
    p2_ref[...] = jnp.concatenate(
        [hw, jnp.ones((ROW_BLK, 128 - D_H), jnp.float32)], axis=1)
    r2_ref[...] = jnp.dot(h, wr_ref[...], preferred_element_type=jnp.float32)


def _combine2_body(s_ref, s1_ref, r_ref, b_ref, o_ref):
    cnt = s1_ref[0, :, D_H:D_H + 1] + s1_ref[1, :, D_H:D_H + 1]
    inv = 1.0 / jnp.maximum(cnt, 1.0)
    summed = s_ref[0, :, :D_H] + s_ref[1, :, :D_H]
    o_ref[...] = summed * inv + b_ref[...] + r_ref[...]


def _proj1(x, W_l, W_r):
    grid = N_NODES // ROW_BLK
    return pl.pallas_call(
        _proj1_body,
        grid=(grid,),
        in_specs=[
            pl.BlockSpec((ROW_BLK, D_IN), lambda i: (i, 0)),
            pl.BlockSpec((D_IN, D_H), lambda i: (0, 0)),
            pl.BlockSpec((D_IN, D_H), lambda i: (0, 0)),
        ],
        out_specs=[
            pl.BlockSpec((ROW_BLK, 128), lambda i: (i, 0)),
            pl.BlockSpec((ROW_BLK, D_H), lambda i: (i, 0)),
        ],
        out_shape=[
            jax.ShapeDtypeStruct((N_NODES, 128), jnp.float32),
            jax.ShapeDtypeStruct((N_NODES, D_H), jnp.float32),
        ],
    )(x, W_l, W_r)


def _combine1_proj2(s1, r1, b1, W_l, W_r):
    grid = N_NODES // ROW_BLK
    return pl.pallas_call(
        _combine1_proj2_body,
        grid=(grid,),
        in_specs=[
            pl.BlockSpec((NC, ROW_BLK, 128), lambda i: (0, i, 0)),
            pl.BlockSpec((ROW_BLK, D_H), lambda i: (i, 0)),
            pl.BlockSpec((1, D_H), lambda i: (0, 0)),
            pl.BlockSpec((D_H, D_H), lambda i: (0, 0)),
            pl.BlockSpec((D_H, D_H), lambda i: (0, 0)),
        ],
        out_specs=[
            pl.BlockSpec((ROW_BLK, 128), lambda i: (i, 0)),
            pl.BlockSpec((ROW_BLK, D_H), lambda i: (i, 0)),
        ],
        out_shape=[
            jax.ShapeDtypeStruct((N_NODES, 128), jnp.float32),
            jax.ShapeDtypeStruct((N_NODES, D_H), jnp.float32),
        ],
    )(s1, r1, b1, W_l, W_r)


def _combine2(s2, s1, r2, b2):
    grid = N_NODES // ROW_BLK
    return pl.pallas_call(
        _combine2_body,
        grid=(grid,),
        in_specs=[
            pl.BlockSpec((NC, ROW_BLK, 128), lambda i: (0, i, 0)),
            pl.BlockSpec((NC, ROW_BLK, 128), lambda i: (0, i, 0)),
            pl.BlockSpec((ROW_BLK, D_H), lambda i: (i, 0)),
            pl.BlockSpec((1, D_H), lambda i: (0, 0)),
        ],
        out_specs=pl.BlockSpec((ROW_BLK, D_H), lambda i: (i, 0)),
        out_shape=jax.ShapeDtypeStruct((N_NODES, D_H), jnp.float32),
    )(s2, s1, r2, b2)


def kernel(x, edge_index, W1_l, b1, W1_r, W2_l, b2, W2_r):
    ei = edge_index.astype(jnp.int32)
    e_w = N_EDGES // NW
    pad_w = E_W_PAD - e_w
    src = jnp.concatenate(
        [ei[0].reshape(NW, e_w),
         jnp.zeros((NW, pad_w + BLK * CH), jnp.int32)], axis=1
    ).reshape(NW, NBLK + 1, BLK, CH)
    dst = jnp.concatenate(
        [ei[1].reshape(NW, e_w),
         jnp.full((NW, pad_w + BLK * CH), N_NODES, jnp.int32)], axis=1
    ).reshape(NW, NBLK + 1, BLK, CH)
    zeros = jnp.zeros((SLAB, 128), jnp.float32)
    b1_2d = b1.reshape(1, D_H)
    b2_2d = b2.reshape(1, D_H)

    p1, r1 = _proj1(x, W1_l, W1_r)
    s1 = _seg_sum(p1, src, dst, zeros)
    p2, r2 = _combine1_proj2(s1, r1, b1_2d, W2_l, W2_r)
    s2 = _seg_sum(p2, src, dst, zeros)
    out = _combine2(s2, s1, r2, b2_2d)
    return out


# speedup vs baseline: 1.3418x; 1.3418x over previous
import jax
import jax.numpy as jnp
from jax import lax
from jax.experimental import pallas as pl
from jax.experimental.pallas import tpu as pltpu
from jax.experimental.pallas import tpu_sc as plsc

N_NODES = 10000
N_EDGES = 320000
D_IN = 128
D_H = 64

NC = 2
NS = 16
NW = NC * NS
NP = 10240
CH = 128
BLK = 8
NBLK = 10
E_W_PAD = NBLK * BLK * CH
SLAB = NP // NS


def _seg_body(p_hbm, src_hbm, dst_hbm, zeros_hbm, s_out,
              shared_acc, src_v, dst_v, rows_a, rows_b,
              sem_ga, sem_gb, sem_sa, sem_sb):
    cid = lax.axis_index("c")
    sid = lax.axis_index("s")
    w = cid * NS + sid
    slab = pl.multiple_of(sid * SLAB, 8)
    rows = (rows_a, rows_b)
    sem_g = (sem_ga, sem_gb)
    sem_s = (sem_sa, sem_sb)

    pltpu.sync_copy(zeros_hbm, shared_acc.at[pl.ds(slab, SLAB)])
    plsc.subcore_barrier()

    def _blk(blk, _):
        pltpu.sync_copy(src_hbm.at[w, blk], src_v)
        pltpu.sync_copy(dst_hbm.at[w, blk], dst_v)
        g = [None] * BLK
        s = [None] * BLK
        g[0] = pltpu.async_copy(p_hbm.at[src_v.at[0]], rows[0], sem_g[0])
        for b in range(BLK):
            if b + 1 < BLK:
                if b >= 1:
                    s[b - 1].wait()
                g[b + 1] = pltpu.async_copy(p_hbm.at[src_v.at[b + 1]],
                                            rows[(b + 1) % 2],
                                            sem_g[(b + 1) % 2])
            g[b].wait()
            s[b] = pltpu.async_copy(rows[b % 2],
                                    shared_acc.at[dst_v.at[b]],
                                    sem_s[b % 2], add=True)
        s[BLK - 2].wait()
        s[BLK - 1].wait()
        return 0

    lax.fori_loop(0, NBLK, _blk, 0)

    plsc.subcore_barrier()
    pltpu.sync_copy(shared_acc.at[pl.ds(slab, SLAB)],
                    s_out.at[cid, pl.ds(slab, SLAB)])


_seg_sum = pl.kernel(
    _seg_body,
    out_type=jax.ShapeDtypeStruct((NC, NP, 128), jnp.float32),
    mesh=plsc.VectorSubcoreMesh(core_axis_name="c", subcore_axis_name="s"),
    scratch_types=[
        pltpu.VMEM_SHARED((NP, 128), jnp.float32),
        pltpu.VMEM((BLK, CH), jnp.int32),
        pltpu.VMEM((BLK, CH), jnp.int32),
        pltpu.VMEM((CH, 128), jnp.float32),
        pltpu.VMEM((CH, 128), jnp.float32),
        pltpu.SemaphoreType.DMA,
        pltpu.SemaphoreType.DMA,
        pltpu.SemaphoreType.DMA,
        pltpu.SemaphoreType.DMA,
    ],
    name="seg_sum",
)


ROW_BLK = 1000


def _proj1_body(x_ref, wl_ref, wr_ref, p_ref, r_ref):
    xb = x_ref[...]
    pw = jnp.dot(xb, wl_ref[...], preferred_element_type=jnp.float32)
    p_ref[...] = jnp.concatenate(
        [pw, jnp.ones((ROW_BLK, 128 - D_H), jnp.float32)], axis=1)
    r_ref[...] = jnp.dot(xb, wr_ref[...], preferred_element_type=jnp.float32)


def _combine1_proj2_body(s_ref, r_ref, b_ref, wl_ref, wr_ref,
                         p2_ref, r2_ref):
    cnt = s_ref[0, :, D_H:D_H + 1] + s_ref[1, :, D_H:D_H + 1]
    inv = 1.0 / jnp.maximum(cnt, 1.0)
    summed = s_ref[0, :, :D_H] + s_ref[1, :, :D_H]
    h = summed * inv + b_ref[...] + r_ref[...]
    h = jnp.maximum(h, 0.0)
    hw = jnp.dot(h, wl_ref[...], preferred_element_type=jnp.float32)
    p2_ref[...] = jnp.concatenate(
        [hw, jnp.ones((ROW_BLK, 128 - D_H), jnp.float32)], axis=1)
    r2_ref[...] = jnp.dot(h, wr_ref[...], preferred_element_type=jnp.float32)


def _combine2_body(s_ref, s1_ref, r_ref, b_ref, o_ref):
    cnt = s1_ref[0, :, D_H:D_H + 1] + s1_ref[1, :, D_H:D_H + 1]
    inv = 1.0 / jnp.maximum(cnt, 1.0)
    summed = s_ref[0, :, :D_H] + s_ref[1, :, :D_H]
    o_ref[...] = summed * inv + b_ref[...] + r_ref[...]


def _proj1(x, W_l, W_r):
    grid = N_NODES // ROW_BLK
    return pl.pallas_call(
        _proj1_body,
        grid=(grid,),
        in_specs=[
            pl.BlockSpec((ROW_BLK, D_IN), lambda i: (i, 0)),
            pl.BlockSpec((D_IN, D_H), lambda i: (0, 0)),
            pl.BlockSpec((D_IN, D_H), lambda i: (0, 0)),
        ],
        out_specs=[
            pl.BlockSpec((ROW_BLK, 128), lambda i: (i, 0)),
            pl.BlockSpec((ROW_BLK, D_H), lambda i: (i, 0)),
        ],
        out_shape=[
            jax.ShapeDtypeStruct((N_NODES, 128), jnp.float32),
            jax.ShapeDtypeStruct((N_NODES, D_H), jnp.float32),
        ],
    )(x, W_l, W_r)


def _combine1_proj2(s1, r1, b1, W_l, W_r):
    grid = N_NODES // ROW_BLK
    return pl.pallas_call(
        _combine1_proj2_body,
        grid=(grid,),
        in_specs=[
            pl.BlockSpec((NC, ROW_BLK, 128), lambda i: (0, i, 0)),
            pl.BlockSpec((ROW_BLK, D_H), lambda i: (i, 0)),
            pl.BlockSpec((1, D_H), lambda i: (0, 0)),
            pl.BlockSpec((D_H, D_H), lambda i: (0, 0)),
            pl.BlockSpec((D_H, D_H), lambda i: (0, 0)),
        ],
        out_specs=[
            pl.BlockSpec((ROW_BLK, 128), lambda i: (i, 0)),
            pl.BlockSpec((ROW_BLK, D_H), lambda i: (i, 0)),
        ],
        out_shape=[
            jax.ShapeDtypeStruct((N_NODES, 128), jnp.float32),
            jax.ShapeDtypeStruct((N_NODES, D_H), jnp.float32),
        ],
    )(s1, r1, b1, W_l, W_r)


def _combine2(s2, s1, r2, b2):
    grid = N_NODES // ROW_BLK
    return pl.pallas_call(
        _combine2_body,
        grid=(grid,),
        in_specs=[
            pl.BlockSpec((NC, ROW_BLK, 128), lambda i: (0, i, 0)),
            pl.BlockSpec((NC, ROW_BLK, 128), lambda i: (0, i, 0)),
            pl.BlockSpec((ROW_BLK, D_H), lambda i: (i, 0)),
            pl.BlockSpec((1, D_H), lambda i: (0, 0)),
        ],
        out_specs=pl.BlockSpec((ROW_BLK, D_H), lambda i: (i, 0)),
        out_shape=jax.ShapeDtypeStruct((N_NODES, D_H), jnp.float32),
    )(s2, s1, r2, b2)


def kernel(x, edge_index, W1_l, b1, W1_r, W2_l, b2, W2_r):
    ei = edge_index.astype(jnp.int32)
    e_w = N_EDGES // NW
    pad_w = E_W_PAD - e_w
    src = jnp.concatenate(
        [ei[0].reshape(NW, e_w), jnp.zeros((NW, pad_w), jnp.int32)], axis=1
    ).reshape(NW, NBLK, BLK, CH)
    dst = jnp.concatenate(
        [ei[1].reshape(NW, e_w),
         jnp.full((NW, pad_w), N_NODES, jnp.int32)], axis=1
    ).reshape(NW, NBLK, BLK, CH)
    zeros = jnp.zeros((SLAB, 128), jnp.float32)
    b1_2d = b1.reshape(1, D_H)
    b2_2d = b2.reshape(1, D_H)

    p1, r1 = _proj1(x, W1_l, W1_r)
    s1 = _seg_sum(p1, src, dst, zeros)
    p2, r2 = _combine1_proj2(s1, r1, b1_2d, W2_l, W2_r)
    s2 = _seg_sum(p2, src, dst, zeros)
    out = _combine2(s2, s1, r2, b2_2d)
    return out


# speedup vs baseline: 1.3704x; 1.0214x over previous
import jax
import jax.numpy as jnp
from jax import lax
from jax.experimental import pallas as pl
from jax.experimental.pallas import tpu as pltpu
from jax.experimental.pallas import tpu_sc as plsc

N_NODES = 10000
N_EDGES = 320000
D_IN = 128
D_H = 64

NC = 2
NS = 16
NW = NC * NS
NP = 10240
CH = 64
BLK = 10
NBLK = 16
NPAIR = NBLK // 2
DEPTH = 4
E_W_PAD = NBLK * BLK * CH
SLAB = NP // NS


def _seg_body(p_hbm, src_hbm, dst_hbm, zeros_hbm, s_out,
              shared_acc, src_a, dst_a, src_b, dst_b,
              rows_0, rows_1, rows_2, rows_3,
              sem_g0, sem_g1, sem_g2, sem_g3,
              sem_s0, sem_s1, sem_s2, sem_s3, sem_i):
    cid = lax.axis_index("c")
    sid = lax.axis_index("s")
    w = cid * NS + sid
    slab = pl.multiple_of(sid * SLAB, 8)
    rows = (rows_0, rows_1, rows_2, rows_3)
    sem_g = (sem_g0, sem_g1, sem_g2, sem_g3)
    sem_s = (sem_s0, sem_s1, sem_s2, sem_s3)

    pltpu.sync_copy(zeros_hbm, shared_acc.at[pl.ds(slab, SLAB)])
    plsc.subcore_barrier()

    def _pair(i, _):
        pltpu.sync_copy(src_hbm.at[w, 2 * i], src_a)
        pltpu.sync_copy(dst_hbm.at[w, 2 * i], dst_a)
        n = 2 * BLK
        g = [None] * n
        s = [None] * n
        g[0] = pltpu.async_copy(p_hbm.at[src_a.at[0]], rows[0], sem_g[0])
        hb1 = pltpu.async_copy(src_hbm.at[w, 2 * i + 1], src_b, sem_i)
        hb2 = pltpu.async_copy(dst_hbm.at[w, 2 * i + 1], dst_b, sem_i)
        for j in range(1, DEPTH - 1):
            g[j] = pltpu.async_copy(p_hbm.at[src_a.at[j]], rows[j],
                                    sem_g[j])
        for b in range(n):
            j = b + DEPTH - 1
            if j < n:
                if j >= DEPTH:
                    s[j - DEPTH].wait()
                if j == BLK:
                    hb1.wait()
                    hb2.wait()
                nsrc = src_a.at[j] if j < BLK else src_b.at[j - BLK]
                g[j] = pltpu.async_copy(p_hbm.at[nsrc], rows[j % DEPTH],
                                        sem_g[j % DEPTH])
            g[b].wait()
            ndst = dst_a.at[b] if b < BLK else dst_b.at[b - BLK]
            s[b] = pltpu.async_copy(rows[b % DEPTH], shared_acc.at[ndst],
                                    sem_s[b % DEPTH], add=True)
        for j in range(n - DEPTH, n):
            s[j].wait()
        return 0

    lax.fori_loop(0, NPAIR, _pair, 0)

    plsc.subcore_barrier()
    pltpu.sync_copy(shared_acc.at[pl.ds(slab, SLAB)],
                    s_out.at[cid, pl.ds(slab, SLAB)])


_seg_sum = pl.kernel(
    _seg_body,
    out_type=jax.ShapeDtypeStruct((NC, NP, 128), jnp.float32),
    mesh=plsc.VectorSubcoreMesh(core_axis_name="c", subcore_axis_name="s"),
    scratch_types=[
        pltpu.VMEM_SHARED((NP, 128), jnp.float32),
        pltpu.VMEM((BLK, CH), jnp.int32),
        pltpu.VMEM((BLK, CH), jnp.int32),
        pltpu.VMEM((BLK, CH), jnp.int32),
        pltpu.VMEM((BLK, CH), jnp.int32),
        pltpu.VMEM((CH, 128), jnp.float32),
        pltpu.VMEM((CH, 128), jnp.float32),
        pltpu.VMEM((CH, 128), jnp.float32),
        pltpu.VMEM((CH, 128), jnp.float32),
        pltpu.SemaphoreType.DMA,
        pltpu.SemaphoreType.DMA,
        pltpu.SemaphoreType.DMA,
        pltpu.SemaphoreType.DMA,
        pltpu.SemaphoreType.DMA,
        pltpu.SemaphoreType.DMA,
        pltpu.SemaphoreType.DMA,
        pltpu.SemaphoreType.DMA,
        pltpu.SemaphoreType.DMA,
    ],
    name="seg_sum",
)


ROW_BLK = 1000


def _proj1_body(x_ref, wl_ref, wr_ref, p_ref, r_ref):
    xb = x_ref[...]
    pw = jnp.dot(xb, wl_ref[...], preferred_element_type=jnp.float32)
    p_ref[...] = jnp.concatenate(
        [pw, jnp.ones((ROW_BLK, 128 - D_H), jnp.float32)], axis=1)
    r_ref[...] = jnp.dot(xb, wr_ref[...], preferred_element_type=jnp.float32)


def _combine1_proj2_body(s_ref, r_ref, b_ref, wl_ref, wr_ref,
                         p2_ref, r2_ref):
    cnt = s_ref[0, :, D_H:D_H + 1] + s_ref[1, :, D_H:D_H + 1]
    inv = 1.0 / jnp.maximum(cnt, 1.0)
    summed = s_ref[0, :, :D_H] + s_ref[1, :, :D_H]
    h = summed * inv + b_ref[...] + r_ref[...]
    h = jnp.maximum(h, 0.0)
    hw = jnp.dot(h, wl_ref[...], preferred_element_type=jnp.float32)
    p2_ref[...] = jnp.concatenate(
        [hw, jnp.ones((ROW_BLK, 128 - D_H), jnp.float32)], axis=1)
    r2_ref[...] = jnp.dot(h, wr_ref[...], preferred_element_type=jnp.float32)


def _combine2_body(s_ref, s1_ref, r_ref, b_ref, o_ref):
    cnt = s1_ref[0, :, D_H:D_H + 1] + s1_ref[1, :, D_H:D_H + 1]
    inv = 1.0 / jnp.maximum(cnt, 1.0)
    summed = s_ref[0, :, :D_H] + s_ref[1, :, :D_H]
    o_ref[...] = summed * inv + b_ref[...] + r_ref[...]


def _proj1(x, W_l, W_r):
    grid = N_NODES // ROW_BLK
    return pl.pallas_call(
        _proj1_body,
        grid=(grid,),
        in_specs=[
            pl.BlockSpec((ROW_BLK, D_IN), lambda i: (i, 0)),
            pl.BlockSpec((D_IN, D_H), lambda i: (0, 0)),
            pl.BlockSpec((D_IN, D_H), lambda i: (0, 0)),
        ],
        out_specs=[
            pl.BlockSpec((ROW_BLK, 128), lambda i: (i, 0)),
            pl.BlockSpec((ROW_BLK, D_H), lambda i: (i, 0)),
        ],
        out_shape=[
            jax.ShapeDtypeStruct((N_NODES, 128), jnp.float32),
            jax.ShapeDtypeStruct((N_NODES, D_H), jnp.float32),
        ],
    )(x, W_l, W_r)


def _combine1_proj2(s1, r1, b1, W_l, W_r):
    grid = N_NODES // ROW_BLK
    return pl.pallas_call(
        _combine1_proj2_body,
        grid=(grid,),
        in_specs=[
            pl.BlockSpec((NC, ROW_BLK, 128), lambda i: (0, i, 0)),
            pl.BlockSpec((ROW_BLK, D_H), lambda i: (i, 0)),
            pl.BlockSpec((1, D_H), lambda i: (0, 0)),
            pl.BlockSpec((D_H, D_H), lambda i: (0, 0)),
            pl.BlockSpec((D_H, D_H), lambda i: (0, 0)),
        ],
        out_specs=[
            pl.BlockSpec((ROW_BLK, 128), lambda i: (i, 0)),
            pl.BlockSpec((ROW_BLK, D_H), lambda i: (i, 0)),
        ],
        out_shape=[
            jax.ShapeDtypeStruct((N_NODES, 128), jnp.float32),
            jax.ShapeDtypeStruct((N_NODES, D_H), jnp.float32),
        ],
    )(s1, r1, b1, W_l, W_r)


def _combine2(s2, s1, r2, b2):
    grid = N_NODES // ROW_BLK
    return pl.pallas_call(
        _combine2_body,
        grid=(grid,),
        in_specs=[
            pl.BlockSpec((NC, ROW_BLK, 128), lambda i: (0, i, 0)),
            pl.BlockSpec((NC, ROW_BLK, 128), lambda i: (0, i, 0)),
            pl.BlockSpec((ROW_BLK, D_H), lambda i: (i, 0)),
            pl.BlockSpec((1, D_H), lambda i: (0, 0)),
        ],
        out_specs=pl.BlockSpec((ROW_BLK, D_H), lambda i: (i, 0)),
        out_shape=jax.ShapeDtypeStruct((N_NODES, D_H), jnp.float32),
    )(s2, s1, r2, b2)


def kernel(x, edge_index, W1_l, b1, W1_r, W2_l, b2, W2_r):
    ei = edge_index.astype(jnp.int32)
    e_w = N_EDGES // NW
    pad_w = E_W_PAD - e_w
    src = jnp.concatenate(
        [ei[0].reshape(NW, e_w), jnp.zeros((NW, pad_w), jnp.int32)], axis=1
    ).reshape(NW, NBLK, BLK, CH)
    dst = jnp.concatenate(
        [ei[1].reshape(NW, e_w),
         jnp.full((NW, pad_w), N_NODES, jnp.int32)], axis=1
    ).reshape(NW, NBLK, BLK, CH)
    zeros = jnp.zeros((SLAB, 128), jnp.float32)
    b1_2d = b1.reshape(1, D_H)
    b2_2d = b2.reshape(1, D_H)

    p1, r1 = _proj1(x, W1_l, W1_r)
    s1 = _seg_sum(p1, src, dst, zeros)
    p2, r2 = _combine1_proj2(s1, r1, b1_2d, W2_l, W2_r)
    s2 = _seg_sum(p2, src, dst, zeros)
    out = _combine2(s2, s1, r2, b2_2d)
    return out
